# baseline (speedup 1.0000x reference)
import functools

import jax
import jax.numpy as jnp
from jax import lax
from jax.experimental import pallas as pl
from jax.experimental.pallas import tpu as pltpu
from jax.experimental.pallas import tpu_sc as plsc

NC = 2
NS = 16
NW = NC * NS
LN = 16
CHUNK = 128


def _sc_mesh():
    return plsc.VectorSubcoreMesh(
        core_axis_name="c", subcore_axis_name="s",
        num_cores=NC, num_subcores=NS)


def _zero_stripe(tmpl_v, sh, base_rows, rpt):
    off = 0
    while off < rpt:
        n = min(CHUNK, rpt - off)
        pltpu.sync_copy(tmpl_v.at[pl.ds(0, n)],
                        sh.at[pl.ds(base_rows + off, n)])
        off += n


def _sc_feature_sums(x, src2, dst2, np_rows, kc):
    N, D = x.shape
    rpt = np_rows // NS
    pw = kc * CHUNK

    @functools.partial(
        pl.kernel,
        out_type=jax.ShapeDtypeStruct((NC, np_rows, D), jnp.float32),
        mesh=_sc_mesh(),
        scratch_types=[
            pltpu.VMEM((CHUNK,), jnp.int32),
            pltpu.VMEM((CHUNK,), jnp.int32),
            pltpu.VMEM((CHUNK, D), jnp.float32),
            pltpu.VMEM_SHARED((np_rows, D), jnp.float32),
            pltpu.SemaphoreType.DMA,
        ],
    )
    def body(x_hbm, src_hbm, dst_hbm, agg_hbm, src_v, dst_v, rows_v,
             agg_sh, gsem):
        c = lax.axis_index("c")
        s = lax.axis_index("s")
        wid = s * NC + c

        zero = jnp.zeros((LN,), jnp.float32)

        def init_row(i, _):
            for j in range(D // LN):
                rows_v[i, pl.ds(j * LN, LN)] = zero
            return 0

        lax.fori_loop(0, CHUNK, init_row, 0)

        base_rows = s * rpt
        _zero_stripe(rows_v, agg_sh, base_rows, rpt)
        plsc.subcore_barrier()

        def chunk_body(k, _):
            base = wid * pw + k * CHUNK
            pltpu.sync_copy(src_hbm.at[pl.ds(base, CHUNK)], src_v)
            pltpu.sync_copy(dst_hbm.at[pl.ds(base, CHUNK)], dst_v)
            pltpu.async_copy(x_hbm.at[src_v], rows_v, gsem).wait()
            pltpu.sync_copy(rows_v, agg_sh.at[dst_v], add=True)
            return 0

        lax.fori_loop(0, kc, chunk_body, 0)
        plsc.subcore_barrier()

        pltpu.sync_copy(agg_sh.at[pl.ds(base_rows, rpt)],
                        agg_hbm.at[c, pl.ds(base_rows, rpt)])

    return body(x, src2, dst2)


def _sc_degree_sums(dst2, np_rows, kc, D):
    rpt = np_rows // NS
    WIN = 8

    @functools.partial(
        pl.kernel,
        out_type=jax.ShapeDtypeStruct((NC, np_rows, D), jnp.float32),
        mesh=_sc_mesh(),
        scratch_types=[
            pltpu.VMEM((kc, CHUNK), jnp.int32),
            pltpu.VMEM((CHUNK, D), jnp.float32),
            pltpu.VMEM((CHUNK, D), jnp.float32),
            pltpu.VMEM_SHARED((np_rows, D), jnp.float32),
            pltpu.SemaphoreType.DMA,
        ],
    )
    def body(dst_hbm, deg_hbm, dst_all, ones_v, zeros_v, deg_sh, ssem):
        c = lax.axis_index("c")
        s = lax.axis_index("s")
        wid = s * NC + c

        pltpu.sync_copy(dst_hbm.at[pl.ds(wid * kc, kc)], dst_all)

        zero = jnp.zeros((LN,), jnp.float32)
        one = jnp.ones((LN,), jnp.float32)

        def init_row(i, _):
            for j in range(D // LN):
                ones_v[i, pl.ds(j * LN, LN)] = one
                zeros_v[i, pl.ds(j * LN, LN)] = zero
            return 0

        lax.fori_loop(0, CHUNK, init_row, 0)

        base_rows = s * rpt
        _zero_stripe(zeros_v, deg_sh, base_rows, rpt)
        plsc.subcore_barrier()

        def drain_one():
            pltpu.make_async_copy(ones_v, deg_sh.at[pl.ds(0, CHUNK)],
                                  ssem).wait()

        def fire(k, _):
            pltpu.async_copy(ones_v, deg_sh.at[dst_all.at[k]], ssem, add=True)

            @pl.when(k >= WIN)
            def _():
                drain_one()
            return 0

        lax.fori_loop(0, kc, fire, 0)
        for _ in range(WIN):
            drain_one()
        plsc.subcore_barrier()

        pltpu.sync_copy(deg_sh.at[pl.ds(base_rows, rpt)],
                        deg_hbm.at[c, pl.ds(base_rows, rpt)])

    return body(dst2)


def _tc_combine(agg, deg, x, W, b):
    N, D = x.shape
    BR = 1000
    assert N % BR == 0

    def tc_body(agg_ref, deg_ref, x_ref, w_ref, b_ref, o_ref):
        a = agg_ref[0] + agg_ref[1]
        dg = deg_ref[0] + deg_ref[1]
        m = a / jnp.maximum(dg[:, 0:1], 1.0)
        h = lax.dot_general(m, w_ref[...], (((1,), (1,)), ((), ())),
                            preferred_element_type=jnp.float32)
        o_ref[...] = x_ref[...] + h + b_ref[...]

    return pl.pallas_call(
        tc_body,
        grid=(N // BR,),
        in_specs=[
            pl.BlockSpec((NC, BR, D), lambda i: (0, i, 0)),
            pl.BlockSpec((NC, BR, D), lambda i: (0, i, 0)),
            pl.BlockSpec((BR, D), lambda i: (i, 0)),
            pl.BlockSpec((D, D), lambda i: (0, 0)),
            pl.BlockSpec((1, D), lambda i: (0, 0)),
        ],
        out_specs=pl.BlockSpec((BR, D), lambda i: (i, 0)),
        out_shape=jax.ShapeDtypeStruct((N, D), jnp.float32),
    )(agg, deg, x, W, b.reshape(1, D))


def kernel(x, edge_index, W, b):
    N, D = x.shape
    E = edge_index.shape[1]

    np_rows = (N // (NS * 8) + 1) * (NS * 8)
    unit = NW * CHUNK * 4
    ep = ((E + unit - 1) // unit) * unit
    kc = ep // (NW * CHUNK)
    pad = ep - E
    src = edge_index[0]
    dst = edge_index[1]
    srcp = jnp.concatenate([src, jnp.zeros((pad,), jnp.int32)])
    dstp = jnp.concatenate([dst, jnp.full((pad,), N, jnp.int32)])
    src2 = srcp.reshape(NW * kc, CHUNK)
    dst2 = dstp.reshape(NW * kc, CHUNK)

    agg = _sc_feature_sums(x, srcp, dstp, np_rows, kc)
    deg = _sc_degree_sums(dst2, np_rows, kc, D)
    return _tc_combine(agg, deg, x, W, b)


# speedup vs baseline: 1.4336x; 1.4336x over previous
import functools

import jax
import jax.numpy as jnp
from jax import lax
from jax.experimental import pallas as pl
from jax.experimental.pallas import tpu as pltpu
from jax.experimental.pallas import tpu_sc as plsc

NC = 2
NS = 16
NW = NC * NS
LN = 16
CHUNK = 128


def _sc_mesh():
    return plsc.VectorSubcoreMesh(
        core_axis_name="c", subcore_axis_name="s",
        num_cores=NC, num_subcores=NS)


def _zero_stripe(tmpl_v, sh, base_rows, rpt):
    off = 0
    while off < rpt:
        n = min(CHUNK, rpt - off)
        pltpu.sync_copy(tmpl_v.at[pl.ds(0, n)],
                        sh.at[pl.ds(base_rows + off, n)])
        off += n


def _sc_feature_sums(x, src2, dst2, np_rows, kc):
    N, D = x.shape
    rpt = np_rows // NS
    pw = kc * CHUNK

    @functools.partial(
        pl.kernel,
        out_type=jax.ShapeDtypeStruct((NC, np_rows, D), jnp.float32),
        mesh=_sc_mesh(),
        scratch_types=[
            pltpu.VMEM((CHUNK,), jnp.int32),
            pltpu.VMEM((CHUNK,), jnp.int32),
            pltpu.VMEM((CHUNK, D), jnp.float32),
            pltpu.VMEM_SHARED((np_rows, D), jnp.float32),
            pltpu.SemaphoreType.DMA,
        ],
    )
    def body(x_hbm, src_hbm, dst_hbm, agg_hbm, src_v, dst_v, rows_v,
             agg_sh, gsem):
        c = lax.axis_index("c")
        s = lax.axis_index("s")
        wid = s * NC + c

        zero = jnp.zeros((LN,), jnp.float32)

        def init_row(i, _):
            for j in range(D // LN):
                rows_v[i, pl.ds(j * LN, LN)] = zero
            return 0

        lax.fori_loop(0, CHUNK, init_row, 0)

        base_rows = s * rpt
        _zero_stripe(rows_v, agg_sh, base_rows, rpt)
        plsc.subcore_barrier()

        def chunk_body(k, _):
            base = wid * pw + k * CHUNK
            pltpu.sync_copy(src_hbm.at[pl.ds(base, CHUNK)], src_v)
            pltpu.sync_copy(dst_hbm.at[pl.ds(base, CHUNK)], dst_v)
            pltpu.async_copy(x_hbm.at[src_v], rows_v, gsem).wait()
            pltpu.sync_copy(rows_v, agg_sh.at[dst_v], add=True)
            return 0

        lax.fori_loop(0, kc, chunk_body, 0)
        plsc.subcore_barrier()

        pltpu.sync_copy(agg_sh.at[pl.ds(base_rows, rpt)],
                        agg_hbm.at[c, pl.ds(base_rows, rpt)])

    return body(x, src2, dst2)


def _sc_degree_sums(dst2, np_rows, kc, D):
    rpt = np_rows // NS
    pw = kc * CHUNK

    @functools.partial(
        pl.kernel,
        out_type=jax.ShapeDtypeStruct((NC, np_rows, D), jnp.float32),
        mesh=_sc_mesh(),
        scratch_types=[
            pltpu.VMEM((CHUNK,), jnp.int32),
            pltpu.VMEM((CHUNK,), jnp.int32),
            pltpu.VMEM((CHUNK, D), jnp.float32),
            pltpu.VMEM((CHUNK, D), jnp.float32),
            pltpu.VMEM_SHARED((np_rows, D), jnp.float32),
            pltpu.SemaphoreType.DMA,
        ],
    )
    def body(dst_hbm, deg_hbm, dst_v0, dst_v1, ones_v, zeros_v, deg_sh, ssem):
        c = lax.axis_index("c")
        s = lax.axis_index("s")
        wid = s * NC + c

        zero = jnp.zeros((LN,), jnp.float32)
        one = jnp.ones((LN,), jnp.float32)

        def init_row(i, _):
            for j in range(D // LN):
                ones_v[i, pl.ds(j * LN, LN)] = one
                zeros_v[i, pl.ds(j * LN, LN)] = zero
            return 0

        lax.fori_loop(0, CHUNK, init_row, 0)

        base_rows = s * rpt
        _zero_stripe(zeros_v, deg_sh, base_rows, rpt)
        plsc.subcore_barrier()

        def drain_one():
            pltpu.make_async_copy(ones_v, deg_sh.at[pl.ds(0, CHUNK)],
                                  ssem).wait()

        def fire(k, _):
            b = lax.rem(k, 2)
            base = wid * pw + k * CHUNK

            @pl.when(k >= 2)
            def _():
                drain_one()

            @pl.when(b == 0)
            def _():
                pltpu.sync_copy(dst_hbm.at[pl.ds(base, CHUNK)], dst_v0)
                pltpu.async_copy(ones_v, deg_sh.at[dst_v0], ssem, add=True)

            @pl.when(b == 1)
            def _():
                pltpu.sync_copy(dst_hbm.at[pl.ds(base, CHUNK)], dst_v1)
                pltpu.async_copy(ones_v, deg_sh.at[dst_v1], ssem, add=True)
            return 0

        lax.fori_loop(0, kc, fire, 0)
        drain_one()
        drain_one()
        plsc.subcore_barrier()

        pltpu.sync_copy(deg_sh.at[pl.ds(base_rows, rpt)],
                        deg_hbm.at[c, pl.ds(base_rows, rpt)])

    return body(dst2)


def _tc_combine(agg, deg, x, W, b):
    N, D = x.shape
    BR = 1000
    assert N % BR == 0

    def tc_body(agg_ref, deg_ref, x_ref, w_ref, b_ref, o_ref):
        a = agg_ref[0] + agg_ref[1]
        dg = deg_ref[0] + deg_ref[1]
        m = a / jnp.maximum(dg[:, 0:1], 1.0)
        h = lax.dot_general(m, w_ref[...], (((1,), (1,)), ((), ())),
                            preferred_element_type=jnp.float32)
        o_ref[...] = x_ref[...] + h + b_ref[...]

    return pl.pallas_call(
        tc_body,
        grid=(N // BR,),
        in_specs=[
            pl.BlockSpec((NC, BR, D), lambda i: (0, i, 0)),
            pl.BlockSpec((NC, BR, D), lambda i: (0, i, 0)),
            pl.BlockSpec((BR, D), lambda i: (i, 0)),
            pl.BlockSpec((D, D), lambda i: (0, 0)),
            pl.BlockSpec((1, D), lambda i: (0, 0)),
        ],
        out_specs=pl.BlockSpec((BR, D), lambda i: (i, 0)),
        out_shape=jax.ShapeDtypeStruct((N, D), jnp.float32),
    )(agg, deg, x, W, b.reshape(1, D))


def kernel(x, edge_index, W, b):
    N, D = x.shape
    E = edge_index.shape[1]

    np_rows = (N // (NS * 8) + 1) * (NS * 8)
    unit = NW * CHUNK
    ep = ((E + unit - 1) // unit) * unit
    kc = ep // (NW * CHUNK)
    pad = ep - E
    src = edge_index[0]
    dst = edge_index[1]
    srcp = jnp.concatenate([src, jnp.zeros((pad,), jnp.int32)])
    dstp = jnp.concatenate([dst, jnp.full((pad,), N, jnp.int32)])

    agg = _sc_feature_sums(x, srcp, dstp, np_rows, kc)
    deg = _sc_degree_sums(dstp, np_rows, kc, D)
    return _tc_combine(agg, deg, x, W, b)
